# Initial kernel scaffold; baseline (speedup 1.0000x reference)
import jax
import jax.numpy as jnp
from jax.experimental import pallas as pl


def kernel(x, w1, b1, w2, b2):
    raise NotImplementedError("write your pallas kernel here")


# speedup vs baseline: 1.1156x; 1.1156x over previous
import functools

import jax
import jax.numpy as jnp
from jax.experimental import pallas as pl
from jax.experimental.pallas import tpu as pltpu


def _double_conv_kernel(x_ref, w1_ref, b1_ref, w2_ref, b2_ref, mask_ref,
                        out_ref, xs_ref, col_ref, *, k, padding, W, L, SL,
                        cin, cout):

    if SL > 0:
        zeros = jnp.zeros((cin, SL), jnp.bfloat16)
        xs_ref[:, 0:SL] = zeros
        xs_ref[:, SL + L:SL + L + SL] = zeros
    xs_ref[:, SL:SL + L] = x_ref[0].astype(jnp.bfloat16)

    mask_rows = [mask_ref[kw:kw + 1, :] for kw in range(k)]

    def build_col():
        for kh in range(k):
            for kw in range(k):
                tap = kh * k + kw
                start = SL + (kh - padding) * W + (kw - padding)
                patch = xs_ref[:, start:start + L]
                if kw != padding:
                    patch = patch * mask_rows[kw]
                col_ref[tap * cin:(tap + 1) * cin, :] = patch

    build_col()
    h1 = jnp.dot(w1_ref[...], col_ref[...],
                 preferred_element_type=jnp.float32) + b1_ref[...]
    h1 = jnp.where(h1 > 0, h1, 0.01 * h1)

    xs_ref[:, SL:SL + L] = h1.astype(jnp.bfloat16)
    build_col()
    out = jnp.dot(w2_ref[...], col_ref[...],
                  preferred_element_type=jnp.float32) + b2_ref[...]
    out_ref[0] = out


def kernel(x, w1, b1, w2, b2):
    B, cin, H, W = x.shape
    cout = w2.shape[0]
    k = w1.shape[2]
    padding = (k - 1) // 2
    L = H * W
    K = k * k * cin
    S = padding * W + padding
    SL = ((S + 127) // 128) * 128 if S > 0 else 0

    x_flat = x.reshape(B, cin, L)

    w1k = jnp.transpose(w1, (0, 2, 3, 1)).reshape(cin, K).astype(jnp.bfloat16)
    w2k = jnp.transpose(w2, (0, 2, 3, 1)).reshape(cout, K).astype(jnp.bfloat16)
    b1v = b1.reshape(cin, 1).astype(jnp.float32)
    b2v = b2.reshape(cout, 1).astype(jnp.float32)

    col_idx = jnp.arange(L, dtype=jnp.int32) % W
    mask = jnp.stack(
        [((col_idx + (kw - padding) >= 0) & (col_idx + (kw - padding) < W))
         .astype(jnp.bfloat16) for kw in range(k)], axis=0)

    _kernel_fn = functools.partial(
        _double_conv_kernel, k=k, padding=padding, W=W, L=L, SL=SL,
        cin=cin, cout=cout)

    flops = 2 * B * K * L * (cin + cout)
    bytes_accessed = 4 * (B * cin * L + B * cout * L) \
        + 2 * ((cin + cout) * K + k * L) + 4 * (cin + cout)

    out_flat = pl.pallas_call(
        _kernel_fn,
        out_shape=jax.ShapeDtypeStruct((B, cout, L), jnp.float32),
        grid=(B,),
        in_specs=[
            pl.BlockSpec((1, cin, L), lambda b: (b, 0, 0)),
            pl.BlockSpec((cin, K), lambda b: (0, 0)),
            pl.BlockSpec((cin, 1), lambda b: (0, 0)),
            pl.BlockSpec((cout, K), lambda b: (0, 0)),
            pl.BlockSpec((cout, 1), lambda b: (0, 0)),
            pl.BlockSpec((k, L), lambda b: (0, 0)),
        ],
        out_specs=pl.BlockSpec((1, cout, L), lambda b: (b, 0, 0)),
        scratch_shapes=[
            pltpu.VMEM((cin, SL + L + SL), jnp.bfloat16),
            pltpu.VMEM((K, L), jnp.bfloat16),
        ],
        compiler_params=pltpu.CompilerParams(
            dimension_semantics=("parallel",)),
        cost_estimate=pl.CostEstimate(
            flops=flops, transcendentals=0, bytes_accessed=bytes_accessed),
    )(x_flat, w1k, b1v, w2k, b2v, mask)

    return out_flat.reshape(B, cout, H, W)
